# baseline (speedup 1.0000x reference)
import jax
import jax.numpy as jnp
from jax.experimental import pallas as pl


def _post_body(boxes_ref, ts_ref, out_ref):
    i = pl.program_id(0)
    b = boxes_ref[...]
    cx, cy, w, h = b[..., 0], b[..., 1], b[..., 2], b[..., 3]
    x0 = cx - 0.5 * w
    y0 = cy - 0.5 * h
    x1 = cx + 0.5 * w
    y1 = cy + 0.5 * h
    img_h = ts_ref[i, 0]
    img_w = ts_ref[i, 1]
    out_ref[...] = jnp.stack(
        [x0 * img_w, y0 * img_h, x1 * img_w, y1 * img_h], axis=-1
    )


def kernel(pred_logits, pred_boxes, target_sizes, quer_feat, num_queries):
    B, Q, C = pred_logits.shape
    prob = jax.nn.sigmoid(pred_logits)
    flat = prob.reshape(B, Q * C)
    topk_values, topk_indexes = jax.lax.top_k(flat, 100)
    topk_boxes = topk_indexes // C
    labels = topk_indexes % C
    scores = topk_values
    boxes_g = jnp.take_along_axis(pred_boxes, topk_boxes[:, :, None], axis=1)
    target_boxes = pl.pallas_call(
        _post_body,
        grid=(B,),
        in_specs=[
            pl.BlockSpec((1, 100, 4), lambda i: (i, 0, 0)),
            pl.BlockSpec((8, 2), lambda i: (0, 0)),
        ],
        out_specs=pl.BlockSpec((1, 100, 4), lambda i: (i, 0, 0)),
        out_shape=jax.ShapeDtypeStruct((B, 100, 4), jnp.float32),
    )(boxes_g, target_sizes)
    queries = jnp.take_along_axis(quer_feat, topk_boxes[:, :, None], axis=1)
    return scores, labels, target_boxes, queries


# speedup vs baseline: 2.6650x; 2.6650x over previous
import numpy as np

import jax
import jax.numpy as jnp
from jax import lax
from jax.experimental import pallas as pl
from jax.experimental.pallas import tpu as pltpu
from jax.experimental.pallas import tpu_sc as plsc

B = 8
Q = 5000
C = 91
N = Q * C
NPAD = 455040
CHUNK = NPAD // 4
PIECES = 9
PIECE = CHUNK // PIECES
PVEC = PIECE // 16
NBUCKET = 2048
KSHIFT = 21
CAP = 2048
KOUT = 112
IMIN = np.int32(-2147483648)
IMAX = np.int32(2147483647)


def _ikey_from_f32(v):
    u = lax.bitcast_convert_type(v, jnp.uint32)
    neg = (u >> jnp.uint32(31)) == jnp.uint32(1)
    ukey = jnp.where(neg, ~u, u | jnp.uint32(0x80000000))
    return lax.bitcast_convert_type(ukey ^ jnp.uint32(0x80000000), jnp.int32), ukey


def _body(lg_hbm, boxes_hbm, quer_hbm, ts_hbm,
          scores_hbm, labels_hbm, tbox_hbm, qout_hbm,
          buf0, buf1, hist, ck_own, ci_own,
          ckv0, ckv1, ckv2, ckv3, civ0, civ1, civ2, civ3,
          t0, t1, t2, t3, thrv, cntv, tsv,
          wk, wi, sc_out, lb_out, gidx, qloc, tbox, qrows,
          sh_hist, sh_thr, sh_ck, sh_ci, sh_cnt,
          sem0, sem1):
    core = lax.axis_index("c")
    sub = lax.axis_index("s")
    b = core * 4 + sub // 4
    slot = sub % 4
    bslot = sub // 4
    base_local = slot * CHUNK
    base = b * NPAD + base_local

    iota = lax.iota(jnp.int32, 16)
    zeros16 = jnp.zeros((16,), jnp.int32)
    ones16 = jnp.ones((16,), jnp.int32)
    lane_off = iota * NBUCKET

    def zh(i, _):
        hist[pl.ds(i * 16, 16)] = zeros16
        return 0
    lax.fori_loop(0, (NBUCKET * 16) // 16, zh, 0)

    def histo_piece(pbuf):
        def bodyA(j, _):
            v = pbuf[pl.ds(j * 16, 16)]
            _, ukey = _ikey_from_f32(v)
            bucket = lax.bitcast_convert_type(ukey >> jnp.uint32(KSHIFT), jnp.int32)
            plsc.addupdate_scatter(hist, [lane_off + bucket], ones16)
            return 0
        lax.fori_loop(0, PVEC, bodyA, 0)

    bufs = (buf0, buf1)
    sems = (sem0, sem1)

    def stream_chunk(process):
        handles = [None, None]
        handles[0] = pltpu.async_copy(
            lg_hbm.at[pl.ds(base, PIECE)], bufs[0], sems[0])
        for p in range(PIECES):
            nxt = p + 1
            if nxt < PIECES:
                handles[nxt % 2] = pltpu.async_copy(
                    lg_hbm.at[pl.ds(base + nxt * PIECE, PIECE)],
                    bufs[nxt % 2], sems[nxt % 2])
            handles[p % 2].wait()
            process(bufs[p % 2], p)

    stream_chunk(lambda pbuf, p: histo_piece(pbuf))

    def lm(jv, _):
        acc = zeros16
        for l in range(16):
            acc = acc + hist[pl.ds(l * NBUCKET + jv * 16, 16)]
        t0[pl.ds(jv * 16, 16)] = acc
        return 0
    lax.fori_loop(0, NBUCKET // 16, lm, 0)
    pltpu.sync_copy(t0, sh_hist.at[sub])
    plsc.subcore_barrier()

    @pl.when(slot == 0)
    def _():
        for r, t in enumerate((t0, t1, t2, t3)):
            pltpu.sync_copy(sh_hist.at[sub + r], t)

        def scan(t_, carry):
            acc, tb, done = carry
            jv = NBUCKET // 16 - 1 - t_
            vec = (t0[pl.ds(jv * 16, 16)] + t1[pl.ds(jv * 16, 16)]
                   + t2[pl.ds(jv * 16, 16)] + t3[pl.ds(jv * 16, 16)])
            rv = lax.rev(vec, (0,))
            cum = acc + plsc.cumsum(rv)
            c15 = jnp.max(cum)
            hit = cum >= 100
            lane = jnp.min(jnp.where(hit, iota, jnp.int32(99)))
            newly = jnp.logical_and(jnp.logical_not(done), c15 >= 100)
            tb = jnp.where(newly, jv * 16 + (15 - lane), tb)
            done = jnp.logical_or(done, c15 >= 100)
            return c15, tb, done

        _, tbucket, _ = lax.fori_loop(
            0, NBUCKET // 16, scan, (jnp.int32(0), jnp.int32(0), False))
        thrv[...] = jnp.zeros((16,), jnp.int32) + tbucket
        pltpu.sync_copy(thrv, sh_thr.at[bslot])
    plsc.subcore_barrier()

    pltpu.sync_copy(sh_thr.at[bslot], thrv)
    tvec = thrv[...]
    thr_u = lax.bitcast_convert_type(tvec, jnp.uint32) << jnp.uint32(KSHIFT)
    ithr = lax.bitcast_convert_type(thr_u ^ jnp.uint32(0x80000000), jnp.int32)

    def phaseB_piece(pbuf, p, cnt):
        pbase = base_local + p * PIECE
        def bodyB(j, cnt):
            v = pbuf[pl.ds(j * 16, 16)]
            ik, _ = _ikey_from_f32(v)
            mask = ik >= ithr
            fidx = (jnp.zeros((16,), jnp.int32) + (pbase + j * 16)) + iota
            plsc.store_compressed(ck_own.at[pl.ds(cnt, 16)], ik, mask=mask)
            plsc.store_compressed(ci_own.at[pl.ds(cnt, 16)], fidx, mask=mask)
            inc = jnp.sum(jnp.where(mask, ones16, zeros16))
            return jnp.minimum(cnt + inc, jnp.int32(CAP - 16))
        return lax.fori_loop(0, PVEC, bodyB, cnt)

    cnt = jnp.int32(0)
    handles = [None, None]
    handles[0] = pltpu.async_copy(
        lg_hbm.at[pl.ds(base, PIECE)], bufs[0], sems[0])
    for p in range(PIECES):
        nxt = p + 1
        if nxt < PIECES:
            handles[nxt % 2] = pltpu.async_copy(
                lg_hbm.at[pl.ds(base + nxt * PIECE, PIECE)],
                bufs[nxt % 2], sems[nxt % 2])
        handles[p % 2].wait()
        cnt = phaseB_piece(bufs[p % 2], p, cnt)

    ck_own[pl.ds(cnt, 16)] = jnp.zeros((16,), jnp.int32) + IMIN
    ci_own[pl.ds(cnt, 16)] = zeros16
    cntv[...] = zeros16 + cnt
    pltpu.sync_copy(ck_own, sh_ck.at[sub])
    pltpu.sync_copy(ci_own, sh_ci.at[sub])
    pltpu.sync_copy(cntv, sh_cnt.at[sub])
    plsc.subcore_barrier()

    @pl.when(slot == 0)
    def _():
        ckvs = (ckv0, ckv1, ckv2, ckv3)
        civs = (civ0, civ1, civ2, civ3)
        nvecs = []
        for r in range(4):
            pltpu.sync_copy(sh_ck.at[sub + r], ckvs[r])
            pltpu.sync_copy(sh_ci.at[sub + r], civs[r])
            pltpu.sync_copy(sh_cnt.at[sub + r], cntv)
            cr = jnp.max(cntv[...])
            nvecs.append((cr + 15) // 16)

        def zw(g, _):
            wk[pl.ds(g * 16, 16)] = zeros16
            wi[pl.ds(g * 16, 16)] = zeros16
            return 0
        lax.fori_loop(0, KOUT // 16, zw, 0)

        imin_v = zeros16 + IMIN
        imax_v = zeros16 + IMAX

        def extract(k_, carry):
            kp, ip = carry
            kp_v = zeros16 + kp
            ip_v = zeros16 + ip
            bk = imin_v
            bi = imax_v
            for r in range(4):
                def inner(j, bc, r=r):
                    bk_, bi_ = bc
                    kx = ckvs[r][pl.ds(j * 16, 16)]
                    ix = civs[r][pl.ds(j * 16, 16)]
                    elig = jnp.logical_or(
                        kx < kp_v, jnp.logical_and(kx == kp_v, ix > ip_v))
                    kx = jnp.where(elig, kx, imin_v)
                    better = jnp.logical_or(
                        kx > bk_, jnp.logical_and(kx == bk_, ix < bi_))
                    return (jnp.where(better, kx, bk_),
                            jnp.where(better, ix, bi_))
                bk, bi = lax.fori_loop(0, nvecs[r], inner, (bk, bi))
            m = jnp.max(bk)
            m_v = zeros16 + m
            mi = jnp.min(jnp.where(bk == m_v, bi, imax_v))
            lane0 = iota == 0
            plsc.store_scatter(wk, [zeros16 + k_], m_v, mask=lane0)
            plsc.store_scatter(wi, [zeros16 + k_], zeros16 + mi, mask=lane0)
            return m, mi

        lax.fori_loop(0, 100, extract, (IMAX, jnp.int32(-1)))

        pltpu.sync_copy(ts_hbm.at[pl.ds(0, 16)], tsv)
        src_lane = (zeros16 + (2 * b + 1)) - (iota & 1)
        scale_v = plsc.load_gather(tsv, [src_lane])

        for g in range(KOUT // 16):
            ik = wk[pl.ds(g * 16, 16)]
            idx = wi[pl.ds(g * 16, 16)]
            uk = lax.bitcast_convert_type(ik, jnp.uint32) ^ jnp.uint32(0x80000000)
            u = jnp.where(ik >= 0, uk ^ jnp.uint32(0x80000000), ~uk)
            logit = lax.bitcast_convert_type(u, jnp.float32)
            sc_out[pl.ds(g * 16, 16)] = 1.0 / (1.0 + jnp.exp(-logit))
            qrow = idx // C
            lb_out[pl.ds(g * 16, 16)] = idx - qrow * C
            qloc[pl.ds(g * 16, 16)] = qrow
            gidx[pl.ds(g * 16, 16)] = qrow + (zeros16 + b * Q)

        hq = pltpu.async_copy(quer_hbm.at[gidx], qrows, sem0)
        pltpu.sync_copy(boxes_hbm.at[pl.ds(b * Q * 4, Q * 4)],
                        hist.at[pl.ds(0, Q * 4)])
        hq.wait()

        for g in range(KOUT * 4 // 16):
            f = zeros16 + g * 16 + iota
            rowv = plsc.load_gather(qloc, [f >> 2])
            cA = f & 1
            ctr = lax.bitcast_convert_type(
                plsc.load_gather(hist, [rowv * 4 + cA]), jnp.float32)
            ext = lax.bitcast_convert_type(
                plsc.load_gather(hist, [rowv * 4 + (cA | 2)]), jnp.float32)
            sgn = jnp.where((f & 2) == 0, jnp.float32(-0.5), jnp.float32(0.5))
            tbox[pl.ds(g * 16, 16)] = (ctr + sgn * ext) * scale_v

        pltpu.sync_copy(sc_out, scores_hbm.at[pl.ds(b * KOUT, KOUT)])
        pltpu.sync_copy(lb_out, labels_hbm.at[pl.ds(b * KOUT, KOUT)])
        pltpu.sync_copy(tbox, tbox_hbm.at[pl.ds(b * KOUT * 4, KOUT * 4)])
        pltpu.sync_copy(qrows, qout_hbm.at[pl.ds(b * KOUT, KOUT)])


@jax.jit
def kernel(pred_logits, pred_boxes, target_sizes, quer_feat, num_queries=100):
    lg = jnp.pad(pred_logits.reshape(B, N), ((0, 0), (0, NPAD - N)),
                 constant_values=jnp.finfo(jnp.float32).min).reshape(B * NPAD)
    boxes_flat = lax.bitcast_convert_type(pred_boxes, jnp.int32).reshape(B * Q * 4)
    quer_flat = quer_feat.reshape(B * Q, 256)
    ts_flat = jnp.pad(target_sizes.reshape(16), (0, 112))

    mesh = plsc.VectorSubcoreMesh(core_axis_name="c", subcore_axis_name="s")
    f32, i32 = jnp.float32, jnp.int32
    run = pl.kernel(
        _body,
        mesh=mesh,
        compiler_params=pltpu.CompilerParams(needs_layout_passes=False),
        out_type=(
            jax.ShapeDtypeStruct((B * KOUT,), f32),
            jax.ShapeDtypeStruct((B * KOUT,), i32),
            jax.ShapeDtypeStruct((B * KOUT * 4,), f32),
            jax.ShapeDtypeStruct((B * KOUT, 256), f32),
        ),
        scratch_types=[
            pltpu.VMEM((PIECE,), f32),
            pltpu.VMEM((PIECE,), f32),
            pltpu.VMEM((NBUCKET * 16,), i32),
            pltpu.VMEM((CAP,), i32),
            pltpu.VMEM((CAP,), i32),
            pltpu.VMEM((CAP,), i32),
            pltpu.VMEM((CAP,), i32),
            pltpu.VMEM((CAP,), i32),
            pltpu.VMEM((CAP,), i32),
            pltpu.VMEM((CAP,), i32),
            pltpu.VMEM((CAP,), i32),
            pltpu.VMEM((CAP,), i32),
            pltpu.VMEM((CAP,), i32),
            pltpu.VMEM((NBUCKET,), i32),
            pltpu.VMEM((NBUCKET,), i32),
            pltpu.VMEM((NBUCKET,), i32),
            pltpu.VMEM((NBUCKET,), i32),
            pltpu.VMEM((16,), i32),
            pltpu.VMEM((16,), i32),
            pltpu.VMEM((16,), f32),
            pltpu.VMEM((KOUT,), i32),
            pltpu.VMEM((KOUT,), i32),
            pltpu.VMEM((KOUT,), f32),
            pltpu.VMEM((KOUT,), i32),
            pltpu.VMEM((KOUT,), i32),
            pltpu.VMEM((KOUT,), i32),
            pltpu.VMEM((KOUT * 4,), f32),
            pltpu.VMEM((KOUT, 256), f32),
            pltpu.VMEM_SHARED((16, NBUCKET), i32),
            pltpu.VMEM_SHARED((4, 16), i32),
            pltpu.VMEM_SHARED((16, CAP), i32),
            pltpu.VMEM_SHARED((16, CAP), i32),
            pltpu.VMEM_SHARED((16, 16), i32),
            pltpu.SemaphoreType.DMA,
            pltpu.SemaphoreType.DMA,
        ],
    )
    scores_p, labels_p, tbox_p, quer_p = run(lg, boxes_flat, quer_flat, ts_flat)
    return (scores_p.reshape(B, KOUT)[:, :100],
            labels_p.reshape(B, KOUT)[:, :100],
            tbox_p.reshape(B, KOUT, 4)[:, :100],
            quer_p.reshape(B, KOUT, 256)[:, :100])
